# baseline (speedup 1.0000x reference)
import functools

import jax
import jax.numpy as jnp
from jax import lax
from jax.experimental import pallas as pl
from jax.experimental.pallas import tpu as pltpu
from jax.experimental.pallas import tpu_sc as plsc

B = 16384
L = 20
D = 16
NC, NS = 2, 16
NW = NC * NS
BPW = B // NW
IDXW = 128
IPW = BPW * L // IDXW
GROUPS = 16
IPG = IPW // GROUPS
RPG = IPG * IDXW
BRPG = RPG // L


def _body(uid_hbm, mid_hbm, utab_hbm, mtab_hbm, par_hbm, out_hbm,
          uidx_v, midx_v, ur_v, mr_v, par_v, out_v, sem_u, sem_m):
    cid = lax.axis_index("c")
    sid = lax.axis_index("s")
    wid = sid * NC + cid

    pltpu.sync_copy(uid_hbm.at[pl.ds(wid * IPW, IPW)], uidx_v)
    pltpu.sync_copy(mid_hbm.at[pl.ds(wid * IPW, IPW)], midx_v)
    pltpu.sync_copy(par_hbm, par_v)
    w16 = par_v[pl.ds(0, D)]
    bias = par_v[pl.ds(D, D)]
    lane = lax.iota(jnp.int32, D)

    def group(g, carry):
        for j in range(IPG):
            pltpu.async_copy(utab_hbm.at[uidx_v.at[g * IPG + j]],
                             ur_v.at[pl.ds(j * IDXW, IDXW)], sem_u)
            pltpu.async_copy(mtab_hbm.at[midx_v.at[g * IPG + j]],
                             mr_v.at[pl.ds(j * IDXW, IDXW)], sem_m)
        pltpu.make_async_copy(utab_hbm.at[pl.ds(0, RPG)], ur_v, sem_u).wait()
        pltpu.make_async_copy(mtab_hbm.at[pl.ds(0, RPG)], mr_v, sem_m).wait()

        for h in range(BRPG // D):
            def row(i, res):
                o = (h * D + i) * L
                acc = ur_v[o] * mr_v[o]
                for l in range(1, L):
                    acc = acc + ur_v[o + l] * mr_v[o + l]
                s = jnp.sum(acc * w16)
                return jnp.where(lane == i, s, res)
            res = lax.fori_loop(0, D, row, jnp.zeros((D,), jnp.float32))
            out_v[pl.ds(g * BRPG + h * D, D)] = res + bias
        return carry

    lax.fori_loop(0, GROUPS, group, 0)
    pltpu.sync_copy(out_v, out_hbm.at[pl.ds(wid * BPW, BPW)])


@jax.jit
def _sc_call(uid2d, mid2d, utab, mtab, par):
    mesh = plsc.VectorSubcoreMesh(core_axis_name="c", subcore_axis_name="s")
    return pl.kernel(
        _body,
        out_type=jax.ShapeDtypeStruct((B,), jnp.float32),
        mesh=mesh,
        compiler_params=pltpu.CompilerParams(
            needs_layout_passes=False, use_tc_tiling_on_sc=False),
        scratch_types=[
            pltpu.VMEM((IPW, IDXW), jnp.int32),
            pltpu.VMEM((IPW, IDXW), jnp.int32),
            pltpu.VMEM((RPG, D), jnp.float32),
            pltpu.VMEM((RPG, D), jnp.float32),
            pltpu.VMEM((32,), jnp.float32),
            pltpu.VMEM((BPW,), jnp.float32),
            pltpu.SemaphoreType.DMA,
            pltpu.SemaphoreType.DMA,
        ],
    )(uid2d, mid2d, utab, mtab, par)


def kernel(user_id, movie_id, user_table, movie_table, fc_w, fc_b):
    uid2d = user_id.reshape(B * L // IDXW, IDXW)
    mid2d = movie_id.reshape(B * L // IDXW, IDXW)
    par = jnp.concatenate([fc_w[0], jnp.full((D,), fc_b[0], jnp.float32)])
    return _sc_call(uid2d, mid2d, user_table, movie_table, par)


# speedup vs baseline: 1.0171x; 1.0171x over previous
import jax
import jax.numpy as jnp
from jax import lax
from jax.experimental import pallas as pl
from jax.experimental.pallas import tpu as pltpu
from jax.experimental.pallas import tpu_sc as plsc

B = 16384
L = 20
D = 16
NC, NS = 2, 16
NW = NC * NS
BPW = B // NW
IDXW = 128
IPW = BPW * L // IDXW
GROUPS = 16
IPG = IPW // GROUPS
RPG = IPG * IDXW
BRPG = RPG // L


def _body(uid_hbm, mid_hbm, utab_hbm, mtab_hbm, par_hbm, out_hbm,
          uidx_v, midx_v, ur0_v, ur1_v, mr0_v, mr1_v, par_v, out_v,
          sem_u0, sem_u1, sem_m0, sem_m1):
    cid = lax.axis_index("c")
    sid = lax.axis_index("s")
    wid = sid * NC + cid

    pltpu.sync_copy(uid_hbm.at[pl.ds(wid * IPW, IPW)], uidx_v)
    pltpu.sync_copy(mid_hbm.at[pl.ds(wid * IPW, IPW)], midx_v)
    pltpu.sync_copy(par_hbm, par_v)
    w16 = par_v[pl.ds(0, D)]
    bias = par_v[pl.ds(D, D)]
    lane = lax.iota(jnp.int32, D)

    urs = (ur0_v, ur1_v)
    mrs = (mr0_v, mr1_v)
    usem = (sem_u0, sem_u1)
    msem = (sem_m0, sem_m1)

    def fire(g, b):
        for j in range(IPG):
            pltpu.async_copy(utab_hbm.at[uidx_v.at[g * IPG + j]],
                             urs[b].at[pl.ds(j * IDXW, IDXW)], usem[b])
            pltpu.async_copy(mtab_hbm.at[midx_v.at[g * IPG + j]],
                             mrs[b].at[pl.ds(j * IDXW, IDXW)], msem[b])

    def drain(b):
        pltpu.make_async_copy(
            utab_hbm.at[pl.ds(0, RPG)], urs[b], usem[b]).wait()
        pltpu.make_async_copy(
            mtab_hbm.at[pl.ds(0, RPG)], mrs[b], msem[b]).wait()

    def compute(g, b):
        for h in range(BRPG // D):
            def row(i, res):
                o = (h * D + i) * L
                acc = urs[b][o] * mrs[b][o]
                for l in range(1, L):
                    acc = acc + urs[b][o + l] * mrs[b][o + l]
                s = jnp.sum(acc * w16)
                return jnp.where(lane == i, s, res)
            res = lax.fori_loop(0, D, row, jnp.zeros((D,), jnp.float32))
            out_v[pl.ds(g * BRPG + h * D, D)] = res + bias

    fire(0, 0)
    fire(1, 1)

    def pair(p, carry):
        for b in range(2):
            g = p * 2 + b
            drain(b)
            compute(g, b)

            @pl.when(g + 2 < GROUPS)
            def _():
                fire(g + 2, b)
        return carry

    lax.fori_loop(0, GROUPS // 2, pair, 0)
    pltpu.sync_copy(out_v, out_hbm.at[pl.ds(wid * BPW, BPW)])


@jax.jit
def _sc_call(uid2d, mid2d, utab, mtab, par):
    mesh = plsc.VectorSubcoreMesh(core_axis_name="c", subcore_axis_name="s")
    return pl.kernel(
        _body,
        out_type=jax.ShapeDtypeStruct((B,), jnp.float32),
        mesh=mesh,
        compiler_params=pltpu.CompilerParams(
            needs_layout_passes=False, use_tc_tiling_on_sc=False),
        scratch_types=[
            pltpu.VMEM((IPW, IDXW), jnp.int32),
            pltpu.VMEM((IPW, IDXW), jnp.int32),
            pltpu.VMEM((RPG, D), jnp.float32),
            pltpu.VMEM((RPG, D), jnp.float32),
            pltpu.VMEM((RPG, D), jnp.float32),
            pltpu.VMEM((RPG, D), jnp.float32),
            pltpu.VMEM((32,), jnp.float32),
            pltpu.VMEM((BPW,), jnp.float32),
            pltpu.SemaphoreType.DMA,
            pltpu.SemaphoreType.DMA,
            pltpu.SemaphoreType.DMA,
            pltpu.SemaphoreType.DMA,
        ],
    )(uid2d, mid2d, utab, mtab, par)


def kernel(user_id, movie_id, user_table, movie_table, fc_w, fc_b):
    uid2d = user_id.reshape(B * L // IDXW, IDXW)
    mid2d = movie_id.reshape(B * L // IDXW, IDXW)
    par = jnp.concatenate([fc_w[0], jnp.full((D,), fc_b[0], jnp.float32)])
    return _sc_call(uid2d, mid2d, user_table, movie_table, par)
